# Initial kernel scaffold; baseline (speedup 1.0000x reference)
import jax
import jax.numpy as jnp
from jax.experimental import pallas as pl


def kernel(x, atomic_numbers, batch_segments, graph_mask, W1, b1, W2, b2):
    raise NotImplementedError("write your pallas kernel here")


# speedup vs baseline: 1.5495x; 1.5495x over previous
import jax
import jax.numpy as jnp
from jax import lax
from jax.experimental import pallas as pl
from jax.experimental.pallas import tpu as pltpu

N = 100000
D = 128
G = 128
B = 4000
NB = N // B


def _body(x_ref, seg_ref, mask_ref, w1_ref, b1_ref, w2_ref, b2_ref,
          energy_ref, total_ref, acc_ref):
    i = pl.program_id(0)

    h = jnp.dot(x_ref[...], w1_ref[...], preferred_element_type=jnp.float32)
    h = h + b1_ref[...]
    h = h * (0.5 * jnp.tanh(0.5 * h) + 0.5)
    e = jnp.dot(h, w2_ref[...], preferred_element_type=jnp.float32)
    e = e + b2_ref[...]

    ids = seg_ref[0]
    cols = lax.broadcasted_iota(jnp.int32, (B, G), 1)
    e_b = jnp.where(ids == cols, e, 0.0)
    part = jnp.sum(e_b.reshape(B // 8, 8, G), axis=0)

    @pl.when(i == 0)
    def _():
        acc_ref[...] = jnp.zeros_like(acc_ref)

    acc_ref[...] += part

    @pl.when(i == NB - 1)
    def _():
        energy = jnp.sum(acc_ref[...], axis=0, keepdims=True)
        energy = jnp.where(mask_ref[...] != 0, energy, 0.0)
        energy_ref[...] = jnp.broadcast_to(energy, (8, G))
        total_ref[...] = (-jnp.sum(energy)).reshape(1, 1)


@jax.jit
def _run(x2, seg3, mask2, W1, b1r, W2, b2r):
    energy8, total = pl.pallas_call(
        _body,
        grid=(NB,),
        in_specs=[
            pl.BlockSpec((B, D), lambda i: (i, 0)),
            pl.BlockSpec((1, B, 1), lambda i: (i, 0, 0)),
            pl.BlockSpec((1, G), lambda i: (0, 0)),
            pl.BlockSpec((D, D), lambda i: (0, 0)),
            pl.BlockSpec((1, D), lambda i: (0, 0)),
            pl.BlockSpec((D, 1), lambda i: (0, 0)),
            pl.BlockSpec((1, 1), lambda i: (0, 0)),
        ],
        out_specs=[
            pl.BlockSpec((8, G), lambda i: (0, 0)),
            pl.BlockSpec((1, 1), lambda i: (0, 0)),
        ],
        out_shape=[
            jax.ShapeDtypeStruct((8, G), jnp.float32),
            jax.ShapeDtypeStruct((1, 1), jnp.float32),
        ],
        scratch_shapes=[pltpu.VMEM((8, G), jnp.float32)],
    )(x2, seg3, mask2, W1, b1r, W2, b2r)
    return energy8, total


def kernel(x, atomic_numbers, batch_segments, graph_mask, W1, b1, W2, b2):
    x2 = x.reshape(N, D)
    seg3 = batch_segments.astype(jnp.int32).reshape(NB, B, 1)
    mask2 = graph_mask.astype(jnp.int32).reshape(1, G)
    b1r = b1.reshape(1, D)
    b2r = b2.reshape(1, 1)
    energy8, total = _run(x2, seg3, mask2, W1, b1r, W2, b2r)
    return (total[0, 0], energy8[0])


# speedup vs baseline: 1.7274x; 1.1149x over previous
import functools

import jax
import jax.numpy as jnp
from jax import lax
from jax.experimental import pallas as pl
from jax.experimental.pallas import tpu as pltpu
from jax.experimental.pallas import tpu_sc as plsc

N = 100000
D = 128
G = 128
B = 4000
NB = N // B
NW = 32
C = 3136
N_PAD = C * NW
ACC = 256


def _tc_body(x_ref, w1_ref, b1_ref, w2_ref, b2_ref, e_ref):
    h = jnp.dot(x_ref[...], w1_ref[...], preferred_element_type=jnp.float32)
    h = h + b1_ref[...]
    h = h * (0.5 * jnp.tanh(0.5 * h) + 0.5)
    e = jnp.dot(h, w2_ref[...], preferred_element_type=jnp.float32)
    e_ref[...] = e + b2_ref[...]


@jax.jit
def _tc_mlp(x2, W1, b1r, W2, b2r):
    return pl.pallas_call(
        _tc_body,
        grid=(NB,),
        in_specs=[
            pl.BlockSpec((B, D), lambda i: (i, 0)),
            pl.BlockSpec((D, D), lambda i: (0, 0)),
            pl.BlockSpec((1, D), lambda i: (0, 0)),
            pl.BlockSpec((D, 1), lambda i: (0, 0)),
            pl.BlockSpec((1, 1), lambda i: (0, 0)),
        ],
        out_specs=pl.BlockSpec((B, 1), lambda i: (i, 0)),
        out_shape=jax.ShapeDtypeStruct((N_PAD, 1), jnp.float32),
    )(x2, W1, b1r, W2, b2r)


_sc_mesh = plsc.VectorSubcoreMesh(core_axis_name="c", subcore_axis_name="s")


@functools.partial(
    pl.kernel,
    out_type=jax.ShapeDtypeStruct((2, G), jnp.float32),
    mesh=_sc_mesh,
    scratch_types=[
        pltpu.VMEM((C,), jnp.float32),
        pltpu.VMEM((C,), jnp.int32),
        pltpu.VMEM((ACC,), jnp.float32),
        pltpu.VMEM((G,), jnp.int32),
        pltpu.VMEM((G,), jnp.float32),
        pltpu.VMEM_SHARED((G,), jnp.float32),
    ],
    compiler_params=pltpu.CompilerParams(needs_layout_passes=False),
)
def _sc_segsum(e_hbm, seg_hbm, out_hbm, e_v, seg_v, acc_v, idx_v, zero_v,
               shared_acc):
    cid = lax.axis_index("c")
    sid = lax.axis_index("s")
    wid = sid * 2 + cid
    base = wid * C

    pltpu.sync_copy(e_hbm.at[pl.ds(base, C)], e_v)
    pltpu.sync_copy(seg_hbm.at[pl.ds(base, C)], seg_v)

    lane = lax.iota(jnp.int32, 16)
    zeros16 = jnp.zeros((16,), jnp.float32)
    for i in range(ACC // 16):
        acc_v[pl.ds(i * 16, 16)] = zeros16
    for i in range(G // 16):
        idx_v[pl.ds(i * 16, 16)] = lane + 16 * i
        zero_v[pl.ds(i * 16, 16)] = zeros16

    def body(i, carry):
        s = i * 16
        ids = seg_v[pl.ds(s, 16)]
        vals = e_v[pl.ds(s, 16)]
        plsc.addupdate_scatter(acc_v, [ids], vals)
        return carry

    lax.fori_loop(0, C // 16, body, 0)

    @pl.when(sid == 0)
    def _():
        pltpu.sync_copy(zero_v, shared_acc)

    plsc.subcore_barrier()
    pltpu.sync_copy(acc_v.at[pl.ds(0, G)], shared_acc.at[idx_v], add=True)
    plsc.subcore_barrier()

    @pl.when(sid == 0)
    def _():
        pltpu.sync_copy(shared_acc, out_hbm.at[cid])


def kernel(x, atomic_numbers, batch_segments, graph_mask, W1, b1, W2, b2):
    x2 = x.reshape(N, D)
    b1r = b1.reshape(1, D)
    b2r = b2.reshape(1, 1)
    e = _tc_mlp(x2, W1, b1r, W2, b2r).reshape(N_PAD)
    seg_pad = jnp.concatenate(
        [batch_segments.astype(jnp.int32),
         jnp.full((N_PAD - N,), G, dtype=jnp.int32)])
    partials = _sc_segsum(e, seg_pad)
    energy = jnp.where(graph_mask, partials[0] + partials[1], 0.0)
    return (-jnp.sum(energy), energy)


# speedup vs baseline: 1.8976x; 1.0985x over previous
import functools

import jax
import jax.numpy as jnp
from jax import lax
from jax.experimental import pallas as pl
from jax.experimental.pallas import tpu as pltpu
from jax.experimental.pallas import tpu_sc as plsc

N = 100000
D = 128
G = 128
B = 10000
NB = N // B
NW = 32
C = 3136
N_PAD = C * NW
ACC = 256


def _tc_body(x_ref, w1_ref, b1_ref, w2_ref, b2_ref, e_ref):
    h = jnp.dot(x_ref[...], w1_ref[...], preferred_element_type=jnp.float32)
    h = h + b1_ref[...]
    h = h * (0.5 * jnp.tanh(0.5 * h) + 0.5)
    e = jnp.dot(h, w2_ref[...], preferred_element_type=jnp.float32)
    e_ref[...] = e + b2_ref[...]


@jax.jit
def _tc_mlp(x2, W1, b1r, W2, b2r):
    return pl.pallas_call(
        _tc_body,
        grid=(NB,),
        in_specs=[
            pl.BlockSpec((B, D), lambda i: (i, 0)),
            pl.BlockSpec((D, D), lambda i: (0, 0)),
            pl.BlockSpec((1, D), lambda i: (0, 0)),
            pl.BlockSpec((D, 1), lambda i: (0, 0)),
            pl.BlockSpec((1, 1), lambda i: (0, 0)),
        ],
        out_specs=pl.BlockSpec((B, 1), lambda i: (i, 0)),
        out_shape=jax.ShapeDtypeStruct((N_PAD, 1), jnp.float32),
    )(x2, W1, b1r, W2, b2r)


_sc_mesh = plsc.VectorSubcoreMesh(core_axis_name="c", subcore_axis_name="s")


@functools.partial(
    pl.kernel,
    out_type=jax.ShapeDtypeStruct((2, G), jnp.float32),
    mesh=_sc_mesh,
    scratch_types=[
        pltpu.VMEM((C,), jnp.float32),
        pltpu.VMEM((C,), jnp.int32),
        pltpu.VMEM((ACC,), jnp.float32),
        pltpu.VMEM((G,), jnp.int32),
        pltpu.VMEM((G,), jnp.float32),
        pltpu.VMEM_SHARED((G,), jnp.float32),
    ],
    compiler_params=pltpu.CompilerParams(needs_layout_passes=False),
)
def _sc_segsum(e_hbm, seg_hbm, out_hbm, e_v, seg_v, acc_v, idx_v, zero_v,
               shared_acc):
    cid = lax.axis_index("c")
    sid = lax.axis_index("s")
    wid = sid * 2 + cid
    base = wid * C

    pltpu.sync_copy(e_hbm.at[pl.ds(base, C)], e_v)
    pltpu.sync_copy(seg_hbm.at[pl.ds(base, C)], seg_v)

    lane = lax.iota(jnp.int32, 16)
    zeros16 = jnp.zeros((16,), jnp.float32)
    for i in range(ACC // 16):
        acc_v[pl.ds(i * 16, 16)] = zeros16
    for i in range(G // 16):
        idx_v[pl.ds(i * 16, 16)] = lane + 16 * i
        zero_v[pl.ds(i * 16, 16)] = zeros16

    def body(i, carry):
        s = i * 16
        ids = seg_v[pl.ds(s, 16)]
        vals = e_v[pl.ds(s, 16)]
        plsc.addupdate_scatter(acc_v, [ids], vals)
        return carry

    lax.fori_loop(0, C // 16, body, 0)

    @pl.when(sid == 0)
    def _():
        pltpu.sync_copy(zero_v, shared_acc)

    plsc.subcore_barrier()
    pltpu.sync_copy(acc_v.at[pl.ds(0, G)], shared_acc.at[idx_v], add=True)
    plsc.subcore_barrier()

    @pl.when(sid == 0)
    def _():
        pltpu.sync_copy(shared_acc, out_hbm.at[cid])


def kernel(x, atomic_numbers, batch_segments, graph_mask, W1, b1, W2, b2):
    x2 = x.reshape(N, D)
    b1r = b1.reshape(1, D)
    b2r = b2.reshape(1, 1)
    e = _tc_mlp(x2, W1, b1r, W2, b2r).reshape(N_PAD)
    seg_pad = jnp.concatenate(
        [batch_segments.astype(jnp.int32),
         jnp.full((N_PAD - N,), G, dtype=jnp.int32)])
    partials = _sc_segsum(e, seg_pad)
    energy = jnp.where(graph_mask, partials[0] + partials[1], 0.0)
    return (-jnp.sum(energy), energy)


# speedup vs baseline: 1.9023x; 1.0025x over previous
import functools

import jax
import jax.numpy as jnp
from jax import lax
from jax.experimental import pallas as pl
from jax.experimental.pallas import tpu as pltpu
from jax.experimental.pallas import tpu_sc as plsc

N = 100000
D = 128
G = 128
B = 20000
NB = N // B
NW = 32
C = 3136
N_PAD = C * NW
ACC = 256


def _tc_body(x_ref, w1_ref, b1_ref, w2_ref, b2_ref, e_ref):
    h = jnp.dot(x_ref[...], w1_ref[...], preferred_element_type=jnp.float32)
    h = h + b1_ref[...]
    h = h * (0.5 * jnp.tanh(0.5 * h) + 0.5)
    e = jnp.dot(h, w2_ref[...], preferred_element_type=jnp.float32)
    e_ref[...] = e + b2_ref[...]


@jax.jit
def _tc_mlp(x2, W1, b1r, W2, b2r):
    return pl.pallas_call(
        _tc_body,
        grid=(NB,),
        in_specs=[
            pl.BlockSpec((B, D), lambda i: (i, 0)),
            pl.BlockSpec((D, D), lambda i: (0, 0)),
            pl.BlockSpec((1, D), lambda i: (0, 0)),
            pl.BlockSpec((D, 1), lambda i: (0, 0)),
            pl.BlockSpec((1, 1), lambda i: (0, 0)),
        ],
        out_specs=pl.BlockSpec((B, 1), lambda i: (i, 0)),
        out_shape=jax.ShapeDtypeStruct((N_PAD, 1), jnp.float32),
    )(x2, W1, b1r, W2, b2r)


_sc_mesh = plsc.VectorSubcoreMesh(core_axis_name="c", subcore_axis_name="s")


@functools.partial(
    pl.kernel,
    out_type=jax.ShapeDtypeStruct((2, G), jnp.float32),
    mesh=_sc_mesh,
    scratch_types=[
        pltpu.VMEM((C,), jnp.float32),
        pltpu.VMEM((C,), jnp.int32),
        pltpu.VMEM((ACC,), jnp.float32),
        pltpu.VMEM((G,), jnp.int32),
        pltpu.VMEM((G,), jnp.float32),
        pltpu.VMEM_SHARED((G,), jnp.float32),
    ],
    compiler_params=pltpu.CompilerParams(needs_layout_passes=False),
)
def _sc_segsum(e_hbm, seg_hbm, out_hbm, e_v, seg_v, acc_v, idx_v, zero_v,
               shared_acc):
    cid = lax.axis_index("c")
    sid = lax.axis_index("s")
    wid = sid * 2 + cid
    base = wid * C

    pltpu.sync_copy(e_hbm.at[pl.ds(base, C)], e_v)
    pltpu.sync_copy(seg_hbm.at[pl.ds(base, C)], seg_v)

    lane = lax.iota(jnp.int32, 16)
    zeros16 = jnp.zeros((16,), jnp.float32)
    for i in range(ACC // 16):
        acc_v[pl.ds(i * 16, 16)] = zeros16
    for i in range(G // 16):
        idx_v[pl.ds(i * 16, 16)] = lane + 16 * i
        zero_v[pl.ds(i * 16, 16)] = zeros16

    def body(i, carry):
        s = i * 16
        ids = seg_v[pl.ds(s, 16)]
        vals = e_v[pl.ds(s, 16)]
        plsc.addupdate_scatter(acc_v, [ids], vals)
        return carry

    lax.fori_loop(0, C // 16, body, 0)

    @pl.when(sid == 0)
    def _():
        pltpu.sync_copy(zero_v, shared_acc)

    plsc.subcore_barrier()
    pltpu.sync_copy(acc_v.at[pl.ds(0, G)], shared_acc.at[idx_v], add=True)
    plsc.subcore_barrier()

    @pl.when(sid == 0)
    def _():
        pltpu.sync_copy(shared_acc, out_hbm.at[cid])


def kernel(x, atomic_numbers, batch_segments, graph_mask, W1, b1, W2, b2):
    x2 = x.reshape(N, D)
    b1r = b1.reshape(1, D)
    b2r = b2.reshape(1, 1)
    e = _tc_mlp(x2, W1, b1r, W2, b2r).reshape(N_PAD)
    seg_pad = jnp.concatenate(
        [batch_segments.astype(jnp.int32),
         jnp.full((N_PAD - N,), G, dtype=jnp.int32)])
    partials = _sc_segsum(e, seg_pad)
    energy = jnp.where(graph_mask, partials[0] + partials[1], 0.0)
    return (-jnp.sum(energy), energy)
